# Initial kernel scaffold; baseline (speedup 1.0000x reference)
import functools

import jax
import jax.numpy as jnp
from jax import lax
from jax.experimental import pallas as pl
from jax.experimental.pallas import tpu as pltpu
from jax.experimental.pallas import tpu_sc as plsc

_N = 25000
_E = 150000
_D = 128
_COEF = 0.1
_BN_EPS = 1e-5

_NC = 2
_NT = 16
_CH = 128


@functools.lru_cache(maxsize=None)
def _build_sc_segsum(n, e):
    nch = -(-e // _CH)
    q = ((n + _NC * _NT - 1) // (_NC * _NT) + 7) // 8 * 8
    split = _NT * q
    trash = split
    acc_rows = split + 8
    last = n - split - (_NT - 1) * q
    assert 0 < last <= q and split <= n and e % 8 == 0

    mesh = plsc.VectorSubcoreMesh(core_axis_name="c", subcore_axis_name="s")
    f32 = jnp.float32
    osd = jax.ShapeDtypeStruct((n, _D), f32)

    @functools.partial(
        pl.kernel,
        out_type=(osd, osd, osd),
        mesh=mesh,
        scratch_types=[
            pltpu.VMEM_SHARED((acc_rows, _D), f32),
            pltpu.VMEM((_CH,), jnp.int32),
            pltpu.VMEM((_CH,), jnp.int32),
            pltpu.VMEM((_CH,), jnp.int32),
            pltpu.VMEM((_CH, _D), f32),
            pltpu.VMEM((_CH, _D), f32),
            pltpu.SemaphoreType.DMA,
        ],
    )
    def seg(x0, x1, s101, d101, s021, d021, s110, d110, s030, d030,
            out_a, out_b, out_c, acc, src_v, dst_v, dl_v, rows_v, zbuf, sem):
        c = lax.axis_index("c")
        s = lax.axis_index("s")
        lo = c * split
        hi = jnp.where(c == 0, split, n)
        base = s * q

        def _zrow(r, _):
            for j in range(_D // 16):
                zbuf[r, pl.ds(j * 16, 16)] = jnp.zeros((16,), f32)
            return 0
        lax.fori_loop(0, _CH, _zrow, 0)

        def _scan_edges(xt, st, dt):
            nk = (nch - 1 - s) // _NT + 1

            def body(k, _):
                cidx = s + k * _NT
                start = cidx * _CH
                off = jnp.minimum(start, e - _CH)
                pltpu.sync_copy(st.at[pl.ds(off, _CH)], src_v)
                pltpu.sync_copy(dt.at[pl.ds(off, _CH)], dst_v)
                for j in range(_CH // 16):
                    d = dst_v[pl.ds(j * 16, 16)]
                    eid = off + j * 16 + lax.iota(jnp.int32, 16)
                    ok = (eid >= start) & (d >= lo) & (d < hi)
                    dl_v[pl.ds(j * 16, 16)] = jnp.where(ok, d - lo, trash)
                pltpu.async_copy(xt.at[src_v], rows_v, sem).wait()
                pltpu.sync_copy(rows_v, acc.at[dl_v], add=True)
                return 0

            lax.fori_loop(0, nk, body, 0)

        groups = (
            (((x1, s101, d101),), out_a),
            (((x0, s021, d021),), out_b),
            (((x1, s110, d110), (x0, s030, d030)), out_c),
        )
        for arrays, out in groups:
            nfull = q // _CH
            for k in range(nfull):
                pltpu.sync_copy(zbuf, acc.at[pl.ds(base + k * _CH, _CH)])
            rem = q - nfull * _CH
            if rem:
                pltpu.sync_copy(zbuf.at[pl.ds(0, rem)],
                                acc.at[pl.ds(base + nfull * _CH, rem)])
            plsc.subcore_barrier()
            for xt, st, dt in arrays:
                _scan_edges(xt, st, dt)
            plsc.subcore_barrier()
            ragged = (c == _NC - 1) & (s == _NT - 1)

            @pl.when(jnp.logical_not(ragged))
            def _():
                pltpu.sync_copy(acc.at[pl.ds(base, q)],
                                out.at[pl.ds(lo + base, q)])

            @pl.when(ragged)
            def _():
                pltpu.sync_copy(acc.at[pl.ds(base, last)],
                                out.at[pl.ds(lo + base, last)])

            plsc.subcore_barrier()

    return seg


_R = 1000


def _full(i):
    return (0, 0)


def _rowblk(i):
    return (i, 0)


@functools.lru_cache(maxsize=None)
def _build_tc_type1(n):
    grid = -(-n // _R)

    def body(x1, a, b_, gw1, gb1, gw2, gb2, hw, hb, out, stats):
        i = pl.program_id(0)
        gin = x1[...] + a[...]
        t = jnp.maximum(gin @ gw1[...] + gb1[...], 0.0) @ gw2[...] + gb2[...]
        h = (t + (b_[...] @ hw[...] + hb[...]) * _COEF) * 0.5
        hr = jnp.maximum(h, 0.0)
        out[...] = hr

        @pl.when(i == 0)
        def _():
            stats[...] = jnp.zeros_like(stats)

        stats[0:1, :] += jnp.sum(hr, axis=0, keepdims=True)
        stats[1:2, :] += jnp.sum(hr * hr, axis=0, keepdims=True)

    blk = pl.BlockSpec((_R, _D), _rowblk)
    wblk = pl.BlockSpec((_D, _D), _full)
    bblk = pl.BlockSpec((1, _D), _full)
    return pl.pallas_call(
        body,
        grid=(grid,),
        in_specs=[blk, blk, blk, wblk, bblk, wblk, bblk, wblk, bblk],
        out_specs=[pl.BlockSpec((_R, _D), _rowblk),
                   pl.BlockSpec((8, _D), _full)],
        out_shape=[jax.ShapeDtypeStruct((n, _D), jnp.float32),
                   jax.ShapeDtypeStruct((8, _D), jnp.float32)],
    )


@functools.lru_cache(maxsize=None)
def _build_tc_type0(n):
    grid = -(-n // _R)

    def body(cacc, hw, hb, out, stats):
        i = pl.program_id(0)
        h = (cacc[...] @ hw[...]) * (0.5 * _COEF) + hb[...] * _COEF
        hr = jnp.maximum(h, 0.0)
        out[...] = hr

        @pl.when(i == 0)
        def _():
            stats[...] = jnp.zeros_like(stats)

        stats[0:1, :] += jnp.sum(hr, axis=0, keepdims=True)
        stats[1:2, :] += jnp.sum(hr * hr, axis=0, keepdims=True)

    blk = pl.BlockSpec((_R, _D), _rowblk)
    return pl.pallas_call(
        body,
        grid=(grid,),
        in_specs=[blk, pl.BlockSpec((_D, _D), _full),
                  pl.BlockSpec((1, _D), _full)],
        out_specs=[pl.BlockSpec((_R, _D), _rowblk),
                   pl.BlockSpec((8, _D), _full)],
        out_shape=[jax.ShapeDtypeStruct((n, _D), jnp.float32),
                   jax.ShapeDtypeStruct((8, _D), jnp.float32)],
    )


@functools.lru_cache(maxsize=None)
def _build_tc_norm(n):
    grid = -(-n // _R)
    inv_n = 1.0 / n

    def body(hr, stats, g, b, out):
        st = stats[...]
        m = st[0:1] * inv_n
        v = st[1:2] * inv_n - m * m
        scale = g[...] * lax.rsqrt(v + _BN_EPS)
        out[...] = hr[...] * scale + (b[...] - m * scale)

    blk = pl.BlockSpec((_R, _D), _rowblk)
    return pl.pallas_call(
        body,
        grid=(grid,),
        in_specs=[blk, pl.BlockSpec((8, _D), _full),
                  pl.BlockSpec((1, _D), _full), pl.BlockSpec((1, _D), _full)],
        out_specs=blk,
        out_shape=jax.ShapeDtypeStruct((n, _D), jnp.float32),
    )


def _layer(h0, h1, edges, gw1, gb1, gw2, gb2, hw, hb, bng, bnb):
    seg = _build_sc_segsum(_N, _E)
    a, b_, cacc = seg(h0, h1, *edges)
    r2 = lambda v: v.reshape(1, _D)
    h1r, st1 = _build_tc_type1(_N)(h1, a, b_, gw1, r2(gb1), gw2, r2(gb2),
                                   hw, r2(hb))
    h0r, st0 = _build_tc_type0(_N)(cacc, hw, r2(hb))
    norm = _build_tc_norm(_N)
    h0n = norm(h0r, st0, r2(bng), r2(bnb))
    h1n = norm(h1r, st1, r2(bng), r2(bnb))
    return h0n, h1n


def kernel(x0, x1, ei_101, ei_110, ei_021, ei_030,
           gin0_w1, gin0_b1, gin0_w2, gin0_b2, hl0_w, hl0_b, bn0_g, bn0_b,
           gin1_w1, gin1_b1, gin1_w2, gin1_b2, hl1_w, hl1_b, bn1_g, bn1_b):
    i32 = jnp.int32
    edges = (ei_101[0].astype(i32), ei_101[1].astype(i32),
             ei_021[0].astype(i32), ei_021[1].astype(i32),
             ei_110[0].astype(i32), ei_110[1].astype(i32),
             ei_030[0].astype(i32), ei_030[1].astype(i32))
    h0, h1 = _layer(x0, x1, edges,
                    gin0_w1, gin0_b1, gin0_w2, gin0_b2, hl0_w, hl0_b,
                    bn0_g, bn0_b)
    h0, h1 = _layer(h0, h1, edges,
                    gin1_w1, gin1_b1, gin1_w2, gin1_b2, hl1_w, hl1_b,
                    bn1_g, bn1_b)
    return jnp.concatenate([h0, h1], axis=0)


# speedup vs baseline: 3.2992x; 3.2992x over previous
import functools

import jax
import jax.numpy as jnp
from jax import lax
from jax.experimental import pallas as pl
from jax.experimental.pallas import tpu as pltpu
from jax.experimental.pallas import tpu_sc as plsc

_N = 25000
_E = 150000
_D = 128
_COEF = 0.1
_BN_EPS = 1e-5

_NC = 2
_NT = 16
_CH = 128


@functools.lru_cache(maxsize=None)
def _build_sc_segsum(n, e):
    nch = -(-e // _CH)
    q = ((n + _NC * _NT - 1) // (_NC * _NT) + 7) // 8 * 8
    split = _NT * q
    trash = split
    acc_rows = split + 8
    last = n - split - (_NT - 1) * q
    assert 0 < last <= q and split <= n and e % 8 == 0

    mesh = plsc.VectorSubcoreMesh(core_axis_name="c", subcore_axis_name="s")
    f32 = jnp.float32
    osd = jax.ShapeDtypeStruct((n, _D), f32)

    @functools.partial(
        pl.kernel,
        out_type=(osd, osd, osd),
        mesh=mesh,
        scratch_types=[
            pltpu.VMEM_SHARED((acc_rows, _D), f32),
            pltpu.VMEM((_CH,), jnp.int32),
            pltpu.VMEM((_CH,), jnp.int32),
            pltpu.VMEM((_CH,), jnp.int32),
            pltpu.VMEM((_CH, _D), f32),
            pltpu.SemaphoreType.DMA,
        ],
    )
    def seg(x0, x1, s101, d101, s021, d021, s110, d110, s030, d030,
            out_a, out_b, out_c, acc, src_v, dst_v, dl_v, rows_v, sem):
        c = lax.axis_index("c")
        s = lax.axis_index("s")
        lo = c * split
        hi = jnp.where(c == 0, split, n)
        base = s * q

        def _scan_edges(xt, st, dt):
            nk = (nch - 1 - s) // _NT + 1

            def body(k, _):
                cidx = s + k * _NT
                start = cidx * _CH
                off = jnp.minimum(start, e - _CH)
                pltpu.sync_copy(st.at[pl.ds(off, _CH)], src_v)
                pltpu.sync_copy(dt.at[pl.ds(off, _CH)], dst_v)
                for j in range(_CH // 16):
                    d = dst_v[pl.ds(j * 16, 16)]
                    eid = off + j * 16 + lax.iota(jnp.int32, 16)
                    ok = (eid >= start) & (d >= lo) & (d < hi)
                    dl_v[pl.ds(j * 16, 16)] = jnp.where(ok, d - lo, trash)
                pltpu.async_copy(xt.at[src_v], rows_v, sem).wait()
                pltpu.sync_copy(rows_v, acc.at[dl_v], add=True)
                return 0

            lax.fori_loop(0, nk, body, 0)

        groups = (
            (((x1, s101, d101),), out_a),
            (((x0, s021, d021),), out_b),
            (((x1, s110, d110), (x0, s030, d030)), out_c),
        )
        for arrays, out in groups:
            def _zrow(r, _):
                for j in range(_D // 16):
                    rows_v[r, pl.ds(j * 16, 16)] = jnp.zeros((16,), f32)
                return 0
            lax.fori_loop(0, _CH, _zrow, 0)
            nfull = q // _CH
            for k in range(nfull):
                pltpu.sync_copy(rows_v, acc.at[pl.ds(base + k * _CH, _CH)])
            rem = q - nfull * _CH
            if rem:
                pltpu.sync_copy(rows_v.at[pl.ds(0, rem)],
                                acc.at[pl.ds(base + nfull * _CH, rem)])
            plsc.subcore_barrier()
            for xt, st, dt in arrays:
                _scan_edges(xt, st, dt)
            plsc.subcore_barrier()
            ragged = (c == _NC - 1) & (s == _NT - 1)

            @pl.when(jnp.logical_not(ragged))
            def _():
                pltpu.sync_copy(acc.at[pl.ds(base, q)],
                                out.at[pl.ds(lo + base, q)])

            @pl.when(ragged)
            def _():
                pltpu.sync_copy(acc.at[pl.ds(base, last)],
                                out.at[pl.ds(lo + base, last)])

            plsc.subcore_barrier()

    return seg


_R = 1000


def _full(i):
    return (0, 0)


def _rowblk(i):
    return (i, 0)


@functools.lru_cache(maxsize=None)
def _build_tc_type1(n):
    grid = -(-n // _R)

    def body(x1, a, b_, gw1, gb1, gw2, gb2, hw, hb, out, stats):
        i = pl.program_id(0)
        gin = x1[...] + a[...]
        t = jnp.maximum(gin @ gw1[...] + gb1[...], 0.0) @ gw2[...] + gb2[...]
        h = (t + (b_[...] @ hw[...] + hb[...]) * _COEF) * 0.5
        hr = jnp.maximum(h, 0.0)
        out[...] = hr

        @pl.when(i == 0)
        def _():
            stats[...] = jnp.zeros_like(stats)

        stats[0:1, :] += jnp.sum(hr, axis=0, keepdims=True)
        stats[1:2, :] += jnp.sum(hr * hr, axis=0, keepdims=True)

    blk = pl.BlockSpec((_R, _D), _rowblk)
    wblk = pl.BlockSpec((_D, _D), _full)
    bblk = pl.BlockSpec((1, _D), _full)
    return pl.pallas_call(
        body,
        grid=(grid,),
        in_specs=[blk, blk, blk, wblk, bblk, wblk, bblk, wblk, bblk],
        out_specs=[pl.BlockSpec((_R, _D), _rowblk),
                   pl.BlockSpec((8, _D), _full)],
        out_shape=[jax.ShapeDtypeStruct((n, _D), jnp.float32),
                   jax.ShapeDtypeStruct((8, _D), jnp.float32)],
    )


@functools.lru_cache(maxsize=None)
def _build_tc_type0(n):
    grid = -(-n // _R)

    def body(cacc, hw, hb, out, stats):
        i = pl.program_id(0)
        h = (cacc[...] @ hw[...]) * (0.5 * _COEF) + hb[...] * _COEF
        hr = jnp.maximum(h, 0.0)
        out[...] = hr

        @pl.when(i == 0)
        def _():
            stats[...] = jnp.zeros_like(stats)

        stats[0:1, :] += jnp.sum(hr, axis=0, keepdims=True)
        stats[1:2, :] += jnp.sum(hr * hr, axis=0, keepdims=True)

    blk = pl.BlockSpec((_R, _D), _rowblk)
    return pl.pallas_call(
        body,
        grid=(grid,),
        in_specs=[blk, pl.BlockSpec((_D, _D), _full),
                  pl.BlockSpec((1, _D), _full)],
        out_specs=[pl.BlockSpec((_R, _D), _rowblk),
                   pl.BlockSpec((8, _D), _full)],
        out_shape=[jax.ShapeDtypeStruct((n, _D), jnp.float32),
                   jax.ShapeDtypeStruct((8, _D), jnp.float32)],
    )


@functools.lru_cache(maxsize=None)
def _build_tc_norm(n):
    grid = -(-n // _R)
    inv_n = 1.0 / n

    def body(hr, stats, g, b, out):
        st = stats[...]
        m = st[0:1] * inv_n
        v = st[1:2] * inv_n - m * m
        scale = g[...] * lax.rsqrt(v + _BN_EPS)
        out[...] = hr[...] * scale + (b[...] - m * scale)

    blk = pl.BlockSpec((_R, _D), _rowblk)
    return pl.pallas_call(
        body,
        grid=(grid,),
        in_specs=[blk, pl.BlockSpec((8, _D), _full),
                  pl.BlockSpec((1, _D), _full), pl.BlockSpec((1, _D), _full)],
        out_specs=blk,
        out_shape=jax.ShapeDtypeStruct((n, _D), jnp.float32),
    )


def _layer(h0, h1, edges, gw1, gb1, gw2, gb2, hw, hb, bng, bnb):
    seg = _build_sc_segsum(_N, _E)
    a, b_, cacc = seg(h0, h1, *edges)
    r2 = lambda v: v.reshape(1, _D)
    h1r, st1 = _build_tc_type1(_N)(h1, a, b_, gw1, r2(gb1), gw2, r2(gb2),
                                   hw, r2(hb))
    h0r, st0 = _build_tc_type0(_N)(cacc, hw, r2(hb))
    norm = _build_tc_norm(_N)
    h0n = norm(h0r, st0, r2(bng), r2(bnb))
    h1n = norm(h1r, st1, r2(bng), r2(bnb))
    return h0n, h1n


def kernel(x0, x1, ei_101, ei_110, ei_021, ei_030,
           gin0_w1, gin0_b1, gin0_w2, gin0_b2, hl0_w, hl0_b, bn0_g, bn0_b,
           gin1_w1, gin1_b1, gin1_w2, gin1_b2, hl1_w, hl1_b, bn1_g, bn1_b):
    i32 = jnp.int32
    edges = (ei_101[0].astype(i32), ei_101[1].astype(i32),
             ei_021[0].astype(i32), ei_021[1].astype(i32),
             ei_110[0].astype(i32), ei_110[1].astype(i32),
             ei_030[0].astype(i32), ei_030[1].astype(i32))
    h0, h1 = _layer(x0, x1, edges,
                    gin0_w1, gin0_b1, gin0_w2, gin0_b2, hl0_w, hl0_b,
                    bn0_g, bn0_b)
    h0, h1 = _layer(h0, h1, edges,
                    gin1_w1, gin1_b1, gin1_w2, gin1_b2, hl1_w, hl1_b,
                    bn1_g, bn1_b)
    return jnp.concatenate([h0, h1], axis=0)


# speedup vs baseline: 5.4455x; 1.6506x over previous
import functools

import jax
import jax.numpy as jnp
from jax import lax
from jax.experimental import pallas as pl
from jax.experimental.pallas import tpu as pltpu
from jax.experimental.pallas import tpu_sc as plsc

_N = 25000
_E = 150000
_D = 128
_COEF = 0.1
_BN_EPS = 1e-5

_NC = 2
_NT = 16
_CH = 112


@functools.lru_cache(maxsize=None)
def _build_sc_segsum(n, e):
    nch = -(-e // _CH)
    q = ((n + _NC * _NT - 1) // (_NC * _NT) + 7) // 8 * 8
    split = _NT * q
    trash = split
    acc_rows = split + 8
    last = n - split - (_NT - 1) * q
    assert 0 < last <= q and split <= n and e % 8 == 0

    mesh = plsc.VectorSubcoreMesh(core_axis_name="c", subcore_axis_name="s")
    f32 = jnp.float32
    osd = jax.ShapeDtypeStruct((n, _D), f32)

    @functools.partial(
        pl.kernel,
        out_type=(osd, osd, osd),
        mesh=mesh,
        scratch_types=[
            pltpu.VMEM_SHARED((acc_rows, _D), f32),
            [pltpu.VMEM((_CH,), jnp.int32)] * 2,
            [pltpu.VMEM((_CH,), jnp.int32)] * 2,
            [pltpu.VMEM((_CH,), jnp.int32)] * 2,
            [pltpu.VMEM((_CH, _D), f32)] * 2,
            pltpu.SemaphoreType.DMA,
            pltpu.SemaphoreType.DMA,
        ],
    )
    def seg(x0, x1, s101, d101, s021, d021, s110, d110, s030, d030,
            out_a, out_b, out_c, acc, src_v, dst_v, dl_v, rows_v,
            sem_i, sem_g):
        c = lax.axis_index("c")
        s = lax.axis_index("s")
        lo = c * split
        hi = jnp.where(c == 0, split, n)
        base = s * q

        def _scan_edges(xt, st, dt):
            nk = (nch - 1 - s) // _NT + 1

            def _off(k):
                start = (s + k * _NT) * _CH
                return start, jnp.minimum(start, e - _CH)

            def _issue_idx(k, b):
                _, off = _off(k)
                pltpu.async_copy(st.at[pl.ds(off, _CH)], src_v[b], sem_i)
                pltpu.async_copy(dt.at[pl.ds(off, _CH)], dst_v[b], sem_i)

            def _wait_idx(k, b):
                _, off = _off(k)
                pltpu.make_async_copy(st.at[pl.ds(off, _CH)], src_v[b],
                                      sem_i).wait()
                pltpu.make_async_copy(dt.at[pl.ds(off, _CH)], dst_v[b],
                                      sem_i).wait()

            def _chunk(k, b):
                @pl.when(k > 0)
                def _():
                    pltpu.make_async_copy(xt.at[src_v[1 - b]],
                                          rows_v[1 - b], sem_g).wait()

                @pl.when(k + 1 < nk)
                def _():
                    _issue_idx(k + 1, 1 - b)

                _wait_idx(k, b)
                start, off = _off(k)
                for j in range(_CH // 16):
                    d = dst_v[b][pl.ds(j * 16, 16)]
                    eid = off + j * 16 + lax.iota(jnp.int32, 16)
                    ok = (eid >= start) & (d >= lo) & (d < hi)
                    dl_v[b][pl.ds(j * 16, 16)] = jnp.where(ok, d - lo, trash)

                pltpu.async_copy(xt.at[src_v[b]], rows_v[b], sem_g)

                @pl.when(k > 0)
                def _():
                    pltpu.sync_copy(rows_v[1 - b], acc.at[dl_v[1 - b]],
                                    add=True)

            _issue_idx(0, 0)

            def body(p, _):
                _chunk(2 * p, 0)
                k = 2 * p + 1

                @pl.when(k < nk)
                def _():
                    _chunk(k, 1)

                return 0

            lax.fori_loop(0, (nk + 1) // 2, body, 0)

            for b in range(2):
                @pl.when((nk - 1) % 2 == b)
                def _():
                    pltpu.make_async_copy(xt.at[src_v[b]], rows_v[b],
                                          sem_g).wait()
                    pltpu.sync_copy(rows_v[b], acc.at[dl_v[b]], add=True)

        groups = (
            (((x1, s101, d101),), out_a),
            (((x0, s021, d021),), out_b),
            (((x1, s110, d110), (x0, s030, d030)), out_c),
        )
        for arrays, out in groups:
            def _zrow(r, _):
                for j in range(_D // 16):
                    rows_v[0][r, pl.ds(j * 16, 16)] = jnp.zeros((16,), f32)
                return 0
            lax.fori_loop(0, _CH, _zrow, 0)
            nfull = q // _CH
            for k in range(nfull):
                pltpu.sync_copy(rows_v[0], acc.at[pl.ds(base + k * _CH, _CH)])
            rem = q - nfull * _CH
            if rem:
                pltpu.sync_copy(rows_v[0].at[pl.ds(0, rem)],
                                acc.at[pl.ds(base + nfull * _CH, rem)])
            plsc.subcore_barrier()
            for xt, st, dt in arrays:
                _scan_edges(xt, st, dt)
            plsc.subcore_barrier()
            ragged = (c == _NC - 1) & (s == _NT - 1)

            @pl.when(jnp.logical_not(ragged))
            def _():
                pltpu.sync_copy(acc.at[pl.ds(base, q)],
                                out.at[pl.ds(lo + base, q)])

            @pl.when(ragged)
            def _():
                pltpu.sync_copy(acc.at[pl.ds(base, last)],
                                out.at[pl.ds(lo + base, last)])

            plsc.subcore_barrier()

    return seg


_R = 1000


def _full(i):
    return (0, 0)


def _rowblk(i):
    return (i, 0)


@functools.lru_cache(maxsize=None)
def _build_tc_type1(n):
    grid = -(-n // _R)

    def body(x1, a, b_, gw1, gb1, gw2, gb2, hw, hb, out, stats):
        i = pl.program_id(0)
        gin = x1[...] + a[...]
        t = jnp.maximum(gin @ gw1[...] + gb1[...], 0.0) @ gw2[...] + gb2[...]
        h = (t + (b_[...] @ hw[...] + hb[...]) * _COEF) * 0.5
        hr = jnp.maximum(h, 0.0)
        out[...] = hr

        @pl.when(i == 0)
        def _():
            stats[...] = jnp.zeros_like(stats)

        stats[0:1, :] += jnp.sum(hr, axis=0, keepdims=True)
        stats[1:2, :] += jnp.sum(hr * hr, axis=0, keepdims=True)

    blk = pl.BlockSpec((_R, _D), _rowblk)
    wblk = pl.BlockSpec((_D, _D), _full)
    bblk = pl.BlockSpec((1, _D), _full)
    return pl.pallas_call(
        body,
        grid=(grid,),
        in_specs=[blk, blk, blk, wblk, bblk, wblk, bblk, wblk, bblk],
        out_specs=[pl.BlockSpec((_R, _D), _rowblk),
                   pl.BlockSpec((8, _D), _full)],
        out_shape=[jax.ShapeDtypeStruct((n, _D), jnp.float32),
                   jax.ShapeDtypeStruct((8, _D), jnp.float32)],
    )


@functools.lru_cache(maxsize=None)
def _build_tc_type0(n):
    grid = -(-n // _R)

    def body(cacc, hw, hb, out, stats):
        i = pl.program_id(0)
        h = (cacc[...] @ hw[...]) * (0.5 * _COEF) + hb[...] * _COEF
        hr = jnp.maximum(h, 0.0)
        out[...] = hr

        @pl.when(i == 0)
        def _():
            stats[...] = jnp.zeros_like(stats)

        stats[0:1, :] += jnp.sum(hr, axis=0, keepdims=True)
        stats[1:2, :] += jnp.sum(hr * hr, axis=0, keepdims=True)

    blk = pl.BlockSpec((_R, _D), _rowblk)
    return pl.pallas_call(
        body,
        grid=(grid,),
        in_specs=[blk, pl.BlockSpec((_D, _D), _full),
                  pl.BlockSpec((1, _D), _full)],
        out_specs=[pl.BlockSpec((_R, _D), _rowblk),
                   pl.BlockSpec((8, _D), _full)],
        out_shape=[jax.ShapeDtypeStruct((n, _D), jnp.float32),
                   jax.ShapeDtypeStruct((8, _D), jnp.float32)],
    )


@functools.lru_cache(maxsize=None)
def _build_tc_norm(n):
    grid = -(-n // _R)
    inv_n = 1.0 / n

    def body(hr, stats, g, b, out):
        st = stats[...]
        m = st[0:1] * inv_n
        v = st[1:2] * inv_n - m * m
        scale = g[...] * lax.rsqrt(v + _BN_EPS)
        out[...] = hr[...] * scale + (b[...] - m * scale)

    blk = pl.BlockSpec((_R, _D), _rowblk)
    return pl.pallas_call(
        body,
        grid=(grid,),
        in_specs=[blk, pl.BlockSpec((8, _D), _full),
                  pl.BlockSpec((1, _D), _full), pl.BlockSpec((1, _D), _full)],
        out_specs=blk,
        out_shape=jax.ShapeDtypeStruct((n, _D), jnp.float32),
    )


def _layer(h0, h1, edges, gw1, gb1, gw2, gb2, hw, hb, bng, bnb):
    seg = _build_sc_segsum(_N, _E)
    a, b_, cacc = seg(h0, h1, *edges)
    r2 = lambda v: v.reshape(1, _D)
    h1r, st1 = _build_tc_type1(_N)(h1, a, b_, gw1, r2(gb1), gw2, r2(gb2),
                                   hw, r2(hb))
    h0r, st0 = _build_tc_type0(_N)(cacc, hw, r2(hb))
    norm = _build_tc_norm(_N)
    h0n = norm(h0r, st0, r2(bng), r2(bnb))
    h1n = norm(h1r, st1, r2(bng), r2(bnb))
    return h0n, h1n


def kernel(x0, x1, ei_101, ei_110, ei_021, ei_030,
           gin0_w1, gin0_b1, gin0_w2, gin0_b2, hl0_w, hl0_b, bn0_g, bn0_b,
           gin1_w1, gin1_b1, gin1_w2, gin1_b2, hl1_w, hl1_b, bn1_g, bn1_b):
    i32 = jnp.int32
    edges = (ei_101[0].astype(i32), ei_101[1].astype(i32),
             ei_021[0].astype(i32), ei_021[1].astype(i32),
             ei_110[0].astype(i32), ei_110[1].astype(i32),
             ei_030[0].astype(i32), ei_030[1].astype(i32))
    h0, h1 = _layer(x0, x1, edges,
                    gin0_w1, gin0_b1, gin0_w2, gin0_b2, hl0_w, hl0_b,
                    bn0_g, bn0_b)
    h0, h1 = _layer(h0, h1, edges,
                    gin1_w1, gin1_b1, gin1_w2, gin1_b2, hl1_w, hl1_b,
                    bn1_g, bn1_b)
    return jnp.concatenate([h0, h1], axis=0)


# speedup vs baseline: 5.5408x; 1.0175x over previous
import functools

import jax
import jax.numpy as jnp
from jax import lax
from jax.experimental import pallas as pl
from jax.experimental.pallas import tpu as pltpu
from jax.experimental.pallas import tpu_sc as plsc

_N = 25000
_E = 150000
_D = 128
_COEF = 0.1
_BN_EPS = 1e-5

_NC = 2
_NT = 16
_CH = 112


@functools.lru_cache(maxsize=None)
def _build_sc_segsum(n, e):
    nch = -(-e // _CH)
    q = ((n + _NC * _NT - 1) // (_NC * _NT) + 7) // 8 * 8
    split = _NT * q
    trash = split
    acc_rows = split + 4 * _NT
    last = n - split - (_NT - 1) * q
    assert 0 < last <= q and split <= n and e % 8 == 0

    mesh = plsc.VectorSubcoreMesh(core_axis_name="c", subcore_axis_name="s")
    f32 = jnp.float32
    osd = jax.ShapeDtypeStruct((n, _D), f32)

    @functools.partial(
        pl.kernel,
        out_type=(osd, osd, osd),
        mesh=mesh,
        scratch_types=[
            pltpu.VMEM_SHARED((acc_rows, _D), f32),
            [pltpu.VMEM((_CH,), jnp.int32)] * 2,
            [pltpu.VMEM((_CH,), jnp.int32)] * 2,
            [pltpu.VMEM((_CH,), jnp.int32)] * 2,
            [pltpu.VMEM((_CH, _D), f32)] * 2,
            pltpu.SemaphoreType.DMA,
            pltpu.SemaphoreType.DMA,
            pltpu.SemaphoreType.DMA,
        ],
    )
    def seg(x0, x1, s101, d101, s021, d021, s110, d110, s030, d030,
            out_a, out_b, out_c, acc, src_v, dst_v, dl_v, rows_v,
            sem_i, sem_g, sem_s):
        c = lax.axis_index("c")
        s = lax.axis_index("s")
        lo = c * split
        hi = jnp.where(c == 0, split, n)
        base = s * q

        def _scan_edges(xt, st, dt):
            nk = (nch - 1 - s) // _NT + 1

            def _off(k):
                start = (s + k * _NT) * _CH
                return start, jnp.minimum(start, e - _CH)

            def _issue_idx(k, b):
                _, off = _off(k)
                pltpu.async_copy(st.at[pl.ds(off, _CH)], src_v[b], sem_i)
                pltpu.async_copy(dt.at[pl.ds(off, _CH)], dst_v[b], sem_i)

            def _wait_idx(k, b):
                _, off = _off(k)
                pltpu.make_async_copy(st.at[pl.ds(off, _CH)], src_v[b],
                                      sem_i).wait()
                pltpu.make_async_copy(dt.at[pl.ds(off, _CH)], dst_v[b],
                                      sem_i).wait()

            tr = trash + s * 4 + (lax.iota(jnp.int32, 16) & 3)

            def _chunk(k, b):
                @pl.when(k > 0)
                def _():
                    pltpu.make_async_copy(xt.at[src_v[1 - b]],
                                          rows_v[1 - b], sem_g).wait()

                @pl.when(k > 1)
                def _():
                    pltpu.make_async_copy(rows_v[b], acc.at[dl_v[b]],
                                          sem_s).wait()

                @pl.when(k + 1 < nk)
                def _():
                    _issue_idx(k + 1, 1 - b)

                _wait_idx(k, b)
                start, off = _off(k)
                for j in range(_CH // 16):
                    d = dst_v[b][pl.ds(j * 16, 16)]
                    eid = off + j * 16 + lax.iota(jnp.int32, 16)
                    ok = (eid >= start) & (d >= lo) & (d < hi)
                    dl_v[b][pl.ds(j * 16, 16)] = jnp.where(ok, d - lo, tr)

                pltpu.async_copy(xt.at[src_v[b]], rows_v[b], sem_g)

                @pl.when(k > 0)
                def _():
                    pltpu.async_copy(rows_v[1 - b], acc.at[dl_v[1 - b]],
                                     sem_s, add=True)

            _issue_idx(0, 0)

            def body(p, _):
                _chunk(2 * p, 0)
                k = 2 * p + 1

                @pl.when(k < nk)
                def _():
                    _chunk(k, 1)

                return 0

            lax.fori_loop(0, (nk + 1) // 2, body, 0)

            for b in range(2):
                @pl.when((nk - 1) % 2 == b)
                def _():
                    pltpu.make_async_copy(xt.at[src_v[b]], rows_v[b],
                                          sem_g).wait()
                    pltpu.async_copy(rows_v[b], acc.at[dl_v[b]], sem_s,
                                     add=True)
                    pltpu.make_async_copy(rows_v[1 - b],
                                          acc.at[dl_v[1 - b]], sem_s).wait()
                    pltpu.make_async_copy(rows_v[b], acc.at[dl_v[b]],
                                          sem_s).wait()

        groups = (
            (((x1, s101, d101),), out_a),
            (((x0, s021, d021),), out_b),
            (((x1, s110, d110), (x0, s030, d030)), out_c),
        )
        for arrays, out in groups:
            def _zrow(r, _):
                for j in range(_D // 16):
                    rows_v[0][r, pl.ds(j * 16, 16)] = jnp.zeros((16,), f32)
                return 0
            lax.fori_loop(0, _CH, _zrow, 0)
            nfull = q // _CH
            for k in range(nfull):
                pltpu.sync_copy(rows_v[0], acc.at[pl.ds(base + k * _CH, _CH)])
            rem = q - nfull * _CH
            if rem:
                pltpu.sync_copy(rows_v[0].at[pl.ds(0, rem)],
                                acc.at[pl.ds(base + nfull * _CH, rem)])
            plsc.subcore_barrier()
            for xt, st, dt in arrays:
                _scan_edges(xt, st, dt)
            plsc.subcore_barrier()
            ragged = (c == _NC - 1) & (s == _NT - 1)

            @pl.when(jnp.logical_not(ragged))
            def _():
                pltpu.sync_copy(acc.at[pl.ds(base, q)],
                                out.at[pl.ds(lo + base, q)])

            @pl.when(ragged)
            def _():
                pltpu.sync_copy(acc.at[pl.ds(base, last)],
                                out.at[pl.ds(lo + base, last)])

            plsc.subcore_barrier()

    return seg


_R = 1000


def _full(i):
    return (0, 0)


def _rowblk(i):
    return (i, 0)


@functools.lru_cache(maxsize=None)
def _build_tc_type1(n):
    grid = -(-n // _R)

    def body(x1, a, b_, gw1, gb1, gw2, gb2, hw, hb, out, stats):
        i = pl.program_id(0)
        gin = x1[...] + a[...]
        t = jnp.maximum(gin @ gw1[...] + gb1[...], 0.0) @ gw2[...] + gb2[...]
        h = (t + (b_[...] @ hw[...] + hb[...]) * _COEF) * 0.5
        hr = jnp.maximum(h, 0.0)
        out[...] = hr

        @pl.when(i == 0)
        def _():
            stats[...] = jnp.zeros_like(stats)

        stats[0:1, :] += jnp.sum(hr, axis=0, keepdims=True)
        stats[1:2, :] += jnp.sum(hr * hr, axis=0, keepdims=True)

    blk = pl.BlockSpec((_R, _D), _rowblk)
    wblk = pl.BlockSpec((_D, _D), _full)
    bblk = pl.BlockSpec((1, _D), _full)
    return pl.pallas_call(
        body,
        grid=(grid,),
        in_specs=[blk, blk, blk, wblk, bblk, wblk, bblk, wblk, bblk],
        out_specs=[pl.BlockSpec((_R, _D), _rowblk),
                   pl.BlockSpec((8, _D), _full)],
        out_shape=[jax.ShapeDtypeStruct((n, _D), jnp.float32),
                   jax.ShapeDtypeStruct((8, _D), jnp.float32)],
    )


@functools.lru_cache(maxsize=None)
def _build_tc_type0(n):
    grid = -(-n // _R)

    def body(cacc, hw, hb, out, stats):
        i = pl.program_id(0)
        h = (cacc[...] @ hw[...]) * (0.5 * _COEF) + hb[...] * _COEF
        hr = jnp.maximum(h, 0.0)
        out[...] = hr

        @pl.when(i == 0)
        def _():
            stats[...] = jnp.zeros_like(stats)

        stats[0:1, :] += jnp.sum(hr, axis=0, keepdims=True)
        stats[1:2, :] += jnp.sum(hr * hr, axis=0, keepdims=True)

    blk = pl.BlockSpec((_R, _D), _rowblk)
    return pl.pallas_call(
        body,
        grid=(grid,),
        in_specs=[blk, pl.BlockSpec((_D, _D), _full),
                  pl.BlockSpec((1, _D), _full)],
        out_specs=[pl.BlockSpec((_R, _D), _rowblk),
                   pl.BlockSpec((8, _D), _full)],
        out_shape=[jax.ShapeDtypeStruct((n, _D), jnp.float32),
                   jax.ShapeDtypeStruct((8, _D), jnp.float32)],
    )


@functools.lru_cache(maxsize=None)
def _build_tc_norm(n):
    grid = -(-n // _R)
    inv_n = 1.0 / n

    def body(hr, stats, g, b, out):
        st = stats[...]
        m = st[0:1] * inv_n
        v = st[1:2] * inv_n - m * m
        scale = g[...] * lax.rsqrt(v + _BN_EPS)
        out[...] = hr[...] * scale + (b[...] - m * scale)

    blk = pl.BlockSpec((_R, _D), _rowblk)
    return pl.pallas_call(
        body,
        grid=(grid,),
        in_specs=[blk, pl.BlockSpec((8, _D), _full),
                  pl.BlockSpec((1, _D), _full), pl.BlockSpec((1, _D), _full)],
        out_specs=blk,
        out_shape=jax.ShapeDtypeStruct((n, _D), jnp.float32),
    )


def _layer(h0, h1, edges, gw1, gb1, gw2, gb2, hw, hb, bng, bnb):
    seg = _build_sc_segsum(_N, _E)
    a, b_, cacc = seg(h0, h1, *edges)
    r2 = lambda v: v.reshape(1, _D)
    h1r, st1 = _build_tc_type1(_N)(h1, a, b_, gw1, r2(gb1), gw2, r2(gb2),
                                   hw, r2(hb))
    h0r, st0 = _build_tc_type0(_N)(cacc, hw, r2(hb))
    norm = _build_tc_norm(_N)
    h0n = norm(h0r, st0, r2(bng), r2(bnb))
    h1n = norm(h1r, st1, r2(bng), r2(bnb))
    return h0n, h1n


def kernel(x0, x1, ei_101, ei_110, ei_021, ei_030,
           gin0_w1, gin0_b1, gin0_w2, gin0_b2, hl0_w, hl0_b, bn0_g, bn0_b,
           gin1_w1, gin1_b1, gin1_w2, gin1_b2, hl1_w, hl1_b, bn1_g, bn1_b):
    i32 = jnp.int32
    edges = (ei_101[0].astype(i32), ei_101[1].astype(i32),
             ei_021[0].astype(i32), ei_021[1].astype(i32),
             ei_110[0].astype(i32), ei_110[1].astype(i32),
             ei_030[0].astype(i32), ei_030[1].astype(i32))
    h0, h1 = _layer(x0, x1, edges,
                    gin0_w1, gin0_b1, gin0_w2, gin0_b2, hl0_w, hl0_b,
                    bn0_g, bn0_b)
    h0, h1 = _layer(h0, h1, edges,
                    gin1_w1, gin1_b1, gin1_w2, gin1_b2, hl1_w, hl1_b,
                    bn1_g, bn1_b)
    return jnp.concatenate([h0, h1], axis=0)
